# Initial kernel scaffold; baseline (speedup 1.0000x reference)
import jax
import jax.numpy as jnp
from jax.experimental import pallas as pl


def kernel(features, sample_res, W):
    raise NotImplementedError("write your pallas kernel here")


# speedup vs baseline: 4.3006x; 4.3006x over previous
import functools

import jax
import jax.numpy as jnp
from jax import lax
from jax.experimental import pallas as pl
from jax.experimental.pallas import tpu as pltpu
from jax.experimental.pallas import tpu_sc as plsc

D = 128
K = 25
N = 500 * 50
NC = 2
NS = 16
NW = NC * NS
NB = 784
NPAD = NW * NB
OUTBUF = 392
LANES = 16
KS = 32


def _tree_sum(vals):
    while len(vals) > 1:
        nxt = [vals[i] + vals[i + 1] for i in range(0, len(vals) - 1, 2)]
        if len(vals) % 2:
            nxt.append(vals[-1])
        vals = nxt
    return vals[0]


def _sc_body(table_hbm, idx_hbm, out_hbm, idx_v, buf0, buf1, out_v,
             sem0, sem1):
    wid = lax.axis_index("s") * NC + lax.axis_index("c")
    base = wid * NB
    pltpu.sync_copy(idx_hbm.at[pl.ds(base * KS, NB * KS)], idx_v)

    bufs = (buf0, buf1)
    sems = (sem0, sem1)

    def issue(r, b):
        pltpu.async_copy(table_hbm.at[idx_v.at[pl.ds(r * KS, K)]],
                         bufs[b], sems[b])

    def wait(b):
        pltpu.make_async_copy(table_hbm.at[idx_v.at[pl.ds(0, K)]], bufs[b],
                              sems[b]).wait()

    issue(0, 0)
    issue(1, 1)

    def flush_group(fg, _):
        def pair(p, _):
            for b in range(2):
                r = fg * OUTBUF + p * 2 + b
                wait(b)
                nxt = r + 2

                @pl.when(nxt < NB)
                def _():
                    issue(nxt, b)

                lr = p * 2 + b
                for d in range(D // LANES):
                    sl = pl.ds(d * LANES, LANES)
                    out_v[lr, sl] = _tree_sum(
                        [bufs[b][k, sl] for k in range(K)])
            return 0

        lax.fori_loop(0, OUTBUF // 2, pair, 0)
        pltpu.sync_copy(out_v, out_hbm.at[pl.ds(base + fg * OUTBUF, OUTBUF)])
        return 0

    lax.fori_loop(0, NB // OUTBUF, flush_group, 0)


@functools.partial(
    pl.kernel,
    mesh=plsc.VectorSubcoreMesh(core_axis_name="c", subcore_axis_name="s"),
    out_type=jax.ShapeDtypeStruct((NPAD, D), jnp.float32),
    scratch_types=[
        pltpu.VMEM((NB * KS,), jnp.int32),
        pltpu.VMEM((K, D), jnp.float32),
        pltpu.VMEM((K, D), jnp.float32),
        pltpu.VMEM((OUTBUF, D), jnp.float32),
        pltpu.SemaphoreType.DMA,
        pltpu.SemaphoreType.DMA,
    ],
)
def _sc_gather_sum(table_hbm, idx_hbm, out_hbm, idx_v, buf0, buf1, out_v,
                   sem0, sem1):
    _sc_body(table_hbm, idx_hbm, out_hbm, idx_v, buf0, buf1, out_v,
             sem0, sem1)


TCB = 1568


def _mm_body(x_ref, w_ref, o_ref):
    o_ref[...] = jnp.maximum(
        jnp.dot(x_ref[...] * (1.0 / K), w_ref[...],
                preferred_element_type=jnp.float32),
        0.0,
    )


_tc_matmul_relu = pl.pallas_call(
    _mm_body,
    grid=(NPAD // TCB,),
    in_specs=[
        pl.BlockSpec((TCB, D), lambda i: (i, 0)),
        pl.BlockSpec((D, D), lambda i: (0, 0)),
    ],
    out_specs=pl.BlockSpec((TCB, D), lambda i: (i, 0)),
    out_shape=jax.ShapeDtypeStruct((NPAD, D), jnp.float32),
)


def kernel(features, sample_res, W):
    B, G, _ = sample_res.shape
    idx = sample_res.astype(jnp.int32).reshape(N, K)
    idx = jnp.pad(idx, ((0, NPAD - N), (0, KS - K)))
    neigh_sum = _sc_gather_sum(features, idx.reshape(NPAD * KS))
    out = _tc_matmul_relu(neigh_sum, W)
    return out[:N].reshape(B, G, D)


# speedup vs baseline: 5.4769x; 1.2735x over previous
import functools

import jax
import jax.numpy as jnp
from jax import lax
from jax.experimental import pallas as pl
from jax.experimental.pallas import tpu as pltpu
from jax.experimental.pallas import tpu_sc as plsc

D = 128
K = 25
N = 500 * 50
NC = 2
NS = 16
NW = NC * NS
NB = 784
NPAD = NW * NB
OUTBUF = 392
LANES = 16
KS = 32


def _tree_sum(vals):
    while len(vals) > 1:
        nxt = [vals[i] + vals[i + 1] for i in range(0, len(vals) - 1, 2)]
        if len(vals) % 2:
            nxt.append(vals[-1])
        vals = nxt
    return vals[0]


RB = 4


def _sc_body(table_hbm, idx_hbm, out_hbm, idx_v, bufs, sems, out_v):
    wid = lax.axis_index("s") * NC + lax.axis_index("c")
    base = wid * NB
    pltpu.sync_copy(idx_hbm.at[pl.ds(base * KS, NB * KS)], idx_v)

    def issue(r, b):
        pltpu.async_copy(table_hbm.at[idx_v.at[pl.ds(r * KS, K)]],
                         bufs[b], sems[b])

    def wait(b):
        pltpu.make_async_copy(table_hbm.at[idx_v.at[pl.ds(0, K)]], bufs[b],
                              sems[b]).wait()

    for b in range(RB):
        issue(b, b)

    def flush_group(fg, _):
        def group(p, _):
            for b in range(RB):
                r = fg * OUTBUF + p * RB + b
                wait(b)
                nxt = r + RB

                @pl.when(nxt < NB)
                def _():
                    issue(nxt, b)

                lr = p * RB + b
                for d in range(D // LANES):
                    sl = pl.ds(d * LANES, LANES)
                    out_v[lr, sl] = _tree_sum(
                        [bufs[b][k, sl] for k in range(K)])
            return 0

        lax.fori_loop(0, OUTBUF // RB, group, 0)
        pltpu.sync_copy(out_v, out_hbm.at[pl.ds(base + fg * OUTBUF, OUTBUF)])
        return 0

    lax.fori_loop(0, NB // OUTBUF, flush_group, 0)


@functools.partial(
    pl.kernel,
    mesh=plsc.VectorSubcoreMesh(core_axis_name="c", subcore_axis_name="s"),
    out_type=jax.ShapeDtypeStruct((NPAD, D), jnp.float32),
    scratch_types=(
        [pltpu.VMEM((NB * KS,), jnp.int32)]
        + [pltpu.VMEM((K, D), jnp.float32) for _ in range(RB)]
        + [pltpu.VMEM((OUTBUF, D), jnp.float32)]
        + [pltpu.SemaphoreType.DMA for _ in range(RB)]
    ),
)
def _sc_gather_sum(table_hbm, idx_hbm, out_hbm, idx_v, *rest):
    bufs = rest[:RB]
    out_v = rest[RB]
    sems = rest[RB + 1:]
    _sc_body(table_hbm, idx_hbm, out_hbm, idx_v, bufs, sems, out_v)


TCB = 1568


def _mm_body(x_ref, w_ref, o_ref):
    o_ref[...] = jnp.maximum(
        jnp.dot(x_ref[...] * (1.0 / K), w_ref[...],
                preferred_element_type=jnp.float32),
        0.0,
    )


_tc_matmul_relu = pl.pallas_call(
    _mm_body,
    grid=(NPAD // TCB,),
    in_specs=[
        pl.BlockSpec((TCB, D), lambda i: (i, 0)),
        pl.BlockSpec((D, D), lambda i: (0, 0)),
    ],
    out_specs=pl.BlockSpec((TCB, D), lambda i: (i, 0)),
    out_shape=jax.ShapeDtypeStruct((NPAD, D), jnp.float32),
)


def kernel(features, sample_res, W):
    B, G, _ = sample_res.shape
    idx = sample_res.astype(jnp.int32).reshape(N, K)
    idx = jnp.pad(idx, ((0, NPAD - N), (0, KS - K)))
    neigh_sum = _sc_gather_sum(features, idx.reshape(NPAD * KS))
    out = _tc_matmul_relu(neigh_sum, W)
    return out[:N].reshape(B, G, D)


# speedup vs baseline: 6.7030x; 1.2239x over previous
import functools

import jax
import jax.numpy as jnp
from jax import lax
from jax.experimental import pallas as pl
from jax.experimental.pallas import tpu as pltpu
from jax.experimental.pallas import tpu_sc as plsc

D = 128
K = 25
N = 500 * 50
NC = 2
NS = 16
NW = NC * NS
NB = 784
NPAD = NW * NB
GSZ = 112
NG = NB // GSZ


def _sc_body(table_hbm, idx_hbm, out_hbm, idx_v, bufs, sems, osem):
    wid = lax.axis_index("s") * NC + lax.axis_index("c")
    base = wid * NB
    pltpu.sync_copy(idx_hbm.at[pl.ds(wid * K * NB, K * NB)], idx_v)

    def ids(k, g):
        return idx_v.at[pl.ds(k * NB + g * GSZ, GSZ)]

    for g in range(NG):
        pltpu.async_copy(table_hbm.at[ids(0, g)], bufs[g], sems[g])

    for g in range(NG):
        pltpu.make_async_copy(table_hbm.at[ids(0, g)], bufs[g],
                              sems[g]).wait()
        for k in range(1, K):
            pltpu.async_copy(table_hbm.at[ids(k, g)], bufs[g], sems[g],
                             add=True)

    for g in range(NG):
        for _ in range(K - 1):
            pltpu.make_async_copy(table_hbm.at[ids(0, g)], bufs[g],
                                  sems[g]).wait()
        pltpu.async_copy(bufs[g], out_hbm.at[pl.ds(base + g * GSZ, GSZ)],
                         osem)

    for g in range(NG):
        pltpu.make_async_copy(bufs[0],
                              out_hbm.at[pl.ds(base, GSZ)], osem).wait()


@functools.partial(
    pl.kernel,
    mesh=plsc.VectorSubcoreMesh(core_axis_name="c", subcore_axis_name="s"),
    out_type=jax.ShapeDtypeStruct((NPAD, D), jnp.float32),
    scratch_types=(
        [pltpu.VMEM((K * NB,), jnp.int32)]
        + [pltpu.VMEM((GSZ, D), jnp.float32) for _ in range(NG)]
        + [pltpu.SemaphoreType.DMA for _ in range(NG)]
        + [pltpu.SemaphoreType.DMA]
    ),
)
def _sc_gather_sum(table_hbm, idx_hbm, out_hbm, idx_v, *rest):
    bufs = rest[:NG]
    sems = rest[NG:2 * NG]
    osem = rest[2 * NG]
    _sc_body(table_hbm, idx_hbm, out_hbm, idx_v, bufs, sems, osem)


TCB = 1568


def _mm_body(x_ref, w_ref, o_ref):
    o_ref[...] = jnp.maximum(
        jnp.dot(x_ref[...] * (1.0 / K), w_ref[...],
                preferred_element_type=jnp.float32),
        0.0,
    )


_tc_matmul_relu = pl.pallas_call(
    _mm_body,
    grid=(NPAD // TCB,),
    in_specs=[
        pl.BlockSpec((TCB, D), lambda i: (i, 0)),
        pl.BlockSpec((D, D), lambda i: (0, 0)),
    ],
    out_specs=pl.BlockSpec((TCB, D), lambda i: (i, 0)),
    out_shape=jax.ShapeDtypeStruct((NPAD, D), jnp.float32),
)


def kernel(features, sample_res, W):
    B, G, _ = sample_res.shape
    idx = sample_res.astype(jnp.int32).reshape(N, K)
    idx = jnp.pad(idx, ((0, NPAD - N), (0, 0)))
    idx = idx.reshape(NW, NB, K).transpose(0, 2, 1).reshape(-1)
    neigh_sum = _sc_gather_sum(features, idx)
    out = _tc_matmul_relu(neigh_sum, W)
    return out[:N].reshape(B, G, D)


# speedup vs baseline: 9.2451x; 1.3792x over previous
import functools

import jax
import jax.numpy as jnp
from jax import lax
from jax.experimental import pallas as pl
from jax.experimental.pallas import tpu as pltpu
from jax.experimental.pallas import tpu_sc as plsc

D = 128
K = 25
N = 500 * 50
NC = 2
NS = 16
GSZ = 112
NBATCH = 224
PAIRB = NBATCH // NS
CORE0_BATCHES = 7
NACC = 4
NIDX = 2


def _worker_pipeline(ng, bstart, table_hbm, idx_hbm, out_hbm,
                     idx_t, accs, asems, osems, isems):
    def off_rows(i):
        off = (bstart + i) * GSZ
        return jnp.where(off > N - GSZ, N - GSZ, off)

    def stage(i):
        t = i % NIDX
        off = off_rows(i)

        def per_k(k, _):
            pltpu.async_copy(idx_hbm.at[pl.ds(k * N + off, GSZ)],
                             idx_t[t].at[pl.ds(k * GSZ, GSZ)], isems[t])
            return 0

        lax.fori_loop(0, K, per_k, 0)

    def drain_stage(i):
        t = i % NIDX
        pltpu.make_async_copy(idx_hbm.at[pl.ds(0, K * GSZ)], idx_t[t],
                              isems[t]).wait()

    def k0(i):
        b = i % NACC
        pltpu.async_copy(
            table_hbm.at[idx_t[i % NIDX].at[pl.ds(0, GSZ)]],
            accs[b], asems[b])

    def wait_k0(i):
        b = i % NACC
        pltpu.make_async_copy(
            table_hbm.at[idx_t[i % NIDX].at[pl.ds(0, GSZ)]],
            accs[b], asems[b]).wait()

    def adds(i):
        b = i % NACC
        t = i % NIDX

        def per_k(k, _):
            pltpu.async_copy(
                table_hbm.at[idx_t[t].at[pl.ds(k * GSZ, GSZ)]],
                accs[b], asems[b], add=True)
            return 0

        lax.fori_loop(1, K, per_k, 0)

    def drain_adds(i):
        b = i % NACC

        def per_k(k, _):
            pltpu.make_async_copy(
                table_hbm.at[idx_t[i % NIDX].at[pl.ds(0, GSZ)]],
                accs[b], asems[b]).wait()
            return 0

        lax.fori_loop(1, K, per_k, 0)

    def flush(i):
        b = i % NACC
        pltpu.async_copy(accs[b], out_hbm.at[pl.ds(off_rows(i), GSZ)],
                         osems[b])

    def drain_flush(i):
        b = i % NACC
        pltpu.make_async_copy(accs[b], out_hbm.at[pl.ds(off_rows(i), GSZ)],
                              osems[b]).wait()

    stage(0)
    drain_stage(0)
    k0(0)

    for i in range(ng):
        wait_k0(i)
        adds(i)
        if i >= 1:
            drain_adds(i - 1)
            flush(i - 1)
        if i + 1 < ng:
            stage(i + 1)
            drain_stage(i + 1)
            if i + 1 >= NACC:
                drain_flush(i + 1 - NACC)
            k0(i + 1)

    drain_adds(ng - 1)
    flush(ng - 1)
    for j in range(max(0, ng - NACC + 1), ng):
        drain_flush(j)


def _sc_body(table_hbm, idx_hbm, out_hbm, idx_t0, idx_t1,
             acc0, acc1, acc2, acc3, as0, as1, as2, as3,
             os0, os1, os2, os3, is0, is1):
    c = lax.axis_index("c")
    s = lax.axis_index("s")
    idx_t = (idx_t0, idx_t1)
    accs = (acc0, acc1, acc2, acc3)
    asems = (as0, as1, as2, as3)
    osems = (os0, os1, os2, os3)
    isems = (is0, is1)

    a = CORE0_BATCHES
    b = PAIRB - a

    @pl.when(c == 0)
    def _():
        _worker_pipeline(a, s * a, table_hbm, idx_hbm, out_hbm,
                         idx_t, accs, asems, osems, isems)

    @pl.when(c == 1)
    def _():
        _worker_pipeline(b, NS * a + s * b, table_hbm, idx_hbm, out_hbm,
                         idx_t, accs, asems, osems, isems)


@functools.partial(
    pl.kernel,
    mesh=plsc.VectorSubcoreMesh(core_axis_name="c", subcore_axis_name="s"),
    out_type=jax.ShapeDtypeStruct((N, D), jnp.float32),
    scratch_types=(
        [pltpu.VMEM((K * GSZ,), jnp.int32) for _ in range(NIDX)]
        + [pltpu.VMEM((GSZ, D), jnp.float32) for _ in range(NACC)]
        + [pltpu.SemaphoreType.DMA for _ in range(2 * NACC + NIDX)]
    ),
)
def _sc_gather_sum(table_hbm, idx_hbm, out_hbm, *rest):
    _sc_body(table_hbm, idx_hbm, out_hbm, *rest)


def _tr_body(x_ref, o_ref):
    o_ref[...] = x_ref[...].T


_tc_transpose = pl.pallas_call(
    _tr_body,
    out_shape=jax.ShapeDtypeStruct((K, N), jnp.int32),
)


TCB = 1000


def _mm_body(x_ref, w_ref, o_ref):
    o_ref[...] = jnp.maximum(
        jnp.dot(x_ref[...] * (1.0 / K), w_ref[...],
                preferred_element_type=jnp.float32),
        0.0,
    )


_tc_matmul_relu = pl.pallas_call(
    _mm_body,
    grid=(N // TCB,),
    in_specs=[
        pl.BlockSpec((TCB, D), lambda i: (i, 0)),
        pl.BlockSpec((D, D), lambda i: (0, 0)),
    ],
    out_specs=pl.BlockSpec((TCB, D), lambda i: (i, 0)),
    out_shape=jax.ShapeDtypeStruct((N, D), jnp.float32),
)


def kernel(features, sample_res, W):
    B, G, _ = sample_res.shape
    idx = sample_res.astype(jnp.int32).reshape(N, K)
    idx_t = _tc_transpose(idx).reshape(K * N)
    neigh_sum = _sc_gather_sum(features, idx_t)
    out = _tc_matmul_relu(neigh_sum, W)
    return out.reshape(B, G, D)


# speedup vs baseline: 9.4730x; 1.0247x over previous
import functools

import jax
import jax.numpy as jnp
from jax import lax
from jax.experimental import pallas as pl
from jax.experimental.pallas import tpu as pltpu
from jax.experimental.pallas import tpu_sc as plsc

D = 128
K = 25
N = 500 * 50
NC = 2
NS = 16
GSZ = 112
NBATCH = 224
PAIRB = NBATCH // NS
CORE0_BATCHES = 7
NACC = 4
NIDX = 2


def _worker_pipeline(ng, bstart, table_hbm, idx_hbm, out_hbm,
                     idx_t, accs, asems, osems, isems):
    def off_rows(i):
        off = (bstart + i) * GSZ
        return jnp.where(off > N - GSZ, N - GSZ, off)

    def stage(i):
        t = i % NIDX
        off = off_rows(i)

        def per_k(k, _):
            pltpu.async_copy(idx_hbm.at[pl.ds(k * NPADK + off, GSZ)],
                             idx_t[t].at[pl.ds(k * GSZ, GSZ)], isems[t])
            return 0

        lax.fori_loop(0, K, per_k, 0)

    def drain_stage(i):
        t = i % NIDX
        pltpu.make_async_copy(idx_hbm.at[pl.ds(0, K * GSZ)], idx_t[t],
                              isems[t]).wait()

    def k0(i):
        b = i % NACC
        pltpu.async_copy(
            table_hbm.at[idx_t[i % NIDX].at[pl.ds(0, GSZ)]],
            accs[b], asems[b])

    def wait_k0(i):
        b = i % NACC
        pltpu.make_async_copy(
            table_hbm.at[idx_t[i % NIDX].at[pl.ds(0, GSZ)]],
            accs[b], asems[b]).wait()

    def adds(i):
        b = i % NACC
        t = i % NIDX

        def per_k(k, _):
            pltpu.async_copy(
                table_hbm.at[idx_t[t].at[pl.ds(k * GSZ, GSZ)]],
                accs[b], asems[b], add=True)
            return 0

        lax.fori_loop(1, K, per_k, 0)

    def drain_adds(i):
        b = i % NACC

        def per_k(k, _):
            pltpu.make_async_copy(
                table_hbm.at[idx_t[i % NIDX].at[pl.ds(0, GSZ)]],
                accs[b], asems[b]).wait()
            return 0

        lax.fori_loop(1, K, per_k, 0)

    def flush(i):
        b = i % NACC
        pltpu.async_copy(accs[b], out_hbm.at[pl.ds(off_rows(i), GSZ)],
                         osems[b])

    def drain_flush(i):
        b = i % NACC
        pltpu.make_async_copy(accs[b], out_hbm.at[pl.ds(off_rows(i), GSZ)],
                              osems[b]).wait()

    stage(0)
    drain_stage(0)
    k0(0)

    for i in range(ng):
        wait_k0(i)
        adds(i)
        if i >= 1:
            drain_adds(i - 1)
            flush(i - 1)
        if i + 1 < ng:
            stage(i + 1)
            drain_stage(i + 1)
            if i + 1 >= NACC:
                drain_flush(i + 1 - NACC)
            k0(i + 1)

    drain_adds(ng - 1)
    flush(ng - 1)
    for j in range(max(0, ng - NACC + 1), ng):
        drain_flush(j)


def _sc_body(table_hbm, idx_hbm, out_hbm, idx_t0, idx_t1,
             acc0, acc1, acc2, acc3, as0, as1, as2, as3,
             os0, os1, os2, os3, is0, is1):
    c = lax.axis_index("c")
    s = lax.axis_index("s")
    idx_t = (idx_t0, idx_t1)
    accs = (acc0, acc1, acc2, acc3)
    asems = (as0, as1, as2, as3)
    osems = (os0, os1, os2, os3)
    isems = (is0, is1)

    a = CORE0_BATCHES
    b = PAIRB - a

    @pl.when(c == 0)
    def _():
        _worker_pipeline(a, s * a, table_hbm, idx_hbm, out_hbm,
                         idx_t, accs, asems, osems, isems)

    @pl.when(c == 1)
    def _():
        _worker_pipeline(b, NS * a + s * b, table_hbm, idx_hbm, out_hbm,
                         idx_t, accs, asems, osems, isems)


@functools.partial(
    pl.kernel,
    mesh=plsc.VectorSubcoreMesh(core_axis_name="c", subcore_axis_name="s"),
    out_type=jax.ShapeDtypeStruct((N, D), jnp.float32),
    scratch_types=(
        [pltpu.VMEM((K * GSZ,), jnp.int32) for _ in range(NIDX)]
        + [pltpu.VMEM((GSZ, D), jnp.float32) for _ in range(NACC)]
        + [pltpu.SemaphoreType.DMA for _ in range(2 * NACC + NIDX)]
    ),
)
def _sc_gather_sum(table_hbm, idx_hbm, out_hbm, *rest):
    _sc_body(table_hbm, idx_hbm, out_hbm, *rest)


NPADK = 25600


def _tr_body(x_ref, o_ref):
    xt = x_ref[...].T
    xtp = jnp.concatenate(
        [xt, jnp.zeros((K, NPADK - N), jnp.int32)], axis=1)
    o_ref[...] = xtp.reshape(K, NPADK // D, D)


_tc_transpose = pl.pallas_call(
    _tr_body,
    out_shape=jax.ShapeDtypeStruct((K, NPADK // D, D), jnp.int32),
)


TCB = 1000


def _mm_body(x_ref, w_ref, o_ref):
    o_ref[...] = jnp.maximum(
        jnp.dot(x_ref[...] * (1.0 / K), w_ref[...],
                preferred_element_type=jnp.float32),
        0.0,
    )


_tc_matmul_relu = pl.pallas_call(
    _mm_body,
    grid=(N // TCB,),
    in_specs=[
        pl.BlockSpec((TCB, D), lambda i: (i, 0)),
        pl.BlockSpec((D, D), lambda i: (0, 0)),
    ],
    out_specs=pl.BlockSpec((TCB, D), lambda i: (i, 0)),
    out_shape=jax.ShapeDtypeStruct((N, D), jnp.float32),
)


def kernel(features, sample_res, W):
    B, G, _ = sample_res.shape
    idx = sample_res.astype(jnp.int32).reshape(N, K)
    idx_t = _tc_transpose(idx).reshape(K * NPADK)
    neigh_sum = _sc_gather_sum(features, idx_t)
    out = _tc_matmul_relu(neigh_sum, W)
    return out.reshape(B, G, D)


# speedup vs baseline: 9.7468x; 1.0289x over previous
import functools

import jax
import jax.numpy as jnp
from jax import lax
from jax.experimental import pallas as pl
from jax.experimental.pallas import tpu as pltpu
from jax.experimental.pallas import tpu_sc as plsc

D = 128
K = 25
N = 500 * 50
NC = 2
NS = 16
GSZ = 112
NBATCH = 224
PAIRB = NBATCH // NS
CORE0_BATCHES = 7
NACC = 4
NIDX = 2


def _worker_pipeline(ng, bstart, table_hbm, idx_hbm, out_hbm,
                     idx_t, accs, asems, osems, isems):
    def off_rows(i):
        off = (bstart + i) * GSZ
        return jnp.where(off > N - GSZ, N - GSZ, off)

    def stage(i):
        t = i % NIDX
        off = off_rows(i)

        def per_k(k, _):
            pltpu.async_copy(idx_hbm.at[pl.ds(k * NPADK + off, GSZ)],
                             idx_t[t].at[pl.ds(k * GSZ, GSZ)], isems[t])
            return 0

        lax.fori_loop(0, K, per_k, 0)

    def drain_stage(i):
        t = i % NIDX
        pltpu.make_async_copy(idx_hbm.at[pl.ds(0, K * GSZ)], idx_t[t],
                              isems[t]).wait()

    def k0(i):
        b = i % NACC
        pltpu.async_copy(
            table_hbm.at[idx_t[i % NIDX].at[pl.ds(0, GSZ)]],
            accs[b], asems[b])

    def wait_k0(i):
        b = i % NACC
        pltpu.make_async_copy(
            table_hbm.at[idx_t[i % NIDX].at[pl.ds(0, GSZ)]],
            accs[b], asems[b]).wait()

    def adds(i):
        b = i % NACC
        t = i % NIDX

        def per_k(k, _):
            pltpu.async_copy(
                table_hbm.at[idx_t[t].at[pl.ds(k * GSZ, GSZ)]],
                accs[b], asems[b], add=True)
            return 0

        lax.fori_loop(1, K, per_k, 0)

    def drain_adds(i):
        b = i % NACC

        def per_k(k, _):
            pltpu.make_async_copy(
                table_hbm.at[idx_t[i % NIDX].at[pl.ds(0, GSZ)]],
                accs[b], asems[b]).wait()
            return 0

        lax.fori_loop(1, K, per_k, 0)

    def flush(i):
        b = i % NACC
        pltpu.async_copy(accs[b], out_hbm.at[pl.ds(off_rows(i), GSZ)],
                         osems[b])

    def drain_flush(i):
        b = i % NACC
        pltpu.make_async_copy(accs[b], out_hbm.at[pl.ds(off_rows(i), GSZ)],
                              osems[b]).wait()

    stage(0)
    drain_stage(0)
    k0(0)

    for i in range(ng):
        wait_k0(i)
        adds(i)
        if i >= 1:
            drain_adds(i - 1)
            flush(i - 1)
        if i + 1 < ng:
            stage(i + 1)
            drain_stage(i + 1)
            if i + 1 >= NACC:
                drain_flush(i + 1 - NACC)
            k0(i + 1)

    drain_adds(ng - 1)
    flush(ng - 1)
    for j in range(max(0, ng - NACC + 1), ng):
        drain_flush(j)


def _sc_body(table_hbm, idx_hbm, out_hbm, idx_t0, idx_t1,
             acc0, acc1, acc2, acc3, as0, as1, as2, as3,
             os0, os1, os2, os3, is0, is1):
    c = lax.axis_index("c")
    s = lax.axis_index("s")
    idx_t = (idx_t0, idx_t1)
    accs = (acc0, acc1, acc2, acc3)
    asems = (as0, as1, as2, as3)
    osems = (os0, os1, os2, os3)
    isems = (is0, is1)

    a = CORE0_BATCHES
    b = PAIRB - a

    @pl.when(c == 0)
    def _():
        _worker_pipeline(a, s * a, table_hbm, idx_hbm, out_hbm,
                         idx_t, accs, asems, osems, isems)

    @pl.when(c == 1)
    def _():
        _worker_pipeline(b, NS * a + s * b, table_hbm, idx_hbm, out_hbm,
                         idx_t, accs, asems, osems, isems)


@functools.partial(
    pl.kernel,
    mesh=plsc.VectorSubcoreMesh(core_axis_name="c", subcore_axis_name="s"),
    out_type=jax.ShapeDtypeStruct((N, D), jnp.float32),
    scratch_types=(
        [pltpu.VMEM((K * GSZ,), jnp.int32) for _ in range(NIDX)]
        + [pltpu.VMEM((GSZ, D), jnp.float32) for _ in range(NACC)]
        + [pltpu.SemaphoreType.DMA for _ in range(2 * NACC + NIDX)]
    ),
)
def _sc_gather_sum(table_hbm, idx_hbm, out_hbm, *rest):
    _sc_body(table_hbm, idx_hbm, out_hbm, *rest)


NPADK = 25600


def _tr_body(x_ref, o_ref):
    xt = x_ref[...].reshape(N, K).T
    xtp = jnp.concatenate(
        [xt, jnp.zeros((K, NPADK - N), jnp.int32)], axis=1)
    o_ref[...] = xtp.reshape(K, NPADK // D, D)


_tc_transpose = pl.pallas_call(
    _tr_body,
    out_shape=jax.ShapeDtypeStruct((K, NPADK // D, D), jnp.int32),
)


TCB = 1000


def _mm_body(x_ref, w_ref, o_ref):
    o_ref[...] = jnp.maximum(
        jnp.dot(x_ref[...] * (1.0 / K), w_ref[...],
                preferred_element_type=jnp.float32),
        0.0,
    )


_tc_matmul_relu = pl.pallas_call(
    _mm_body,
    grid=(N // TCB,),
    in_specs=[
        pl.BlockSpec((TCB, D), lambda i: (i, 0)),
        pl.BlockSpec((D, D), lambda i: (0, 0)),
    ],
    out_specs=pl.BlockSpec((TCB, D), lambda i: (i, 0)),
    out_shape=jax.ShapeDtypeStruct((N, D), jnp.float32),
)


def kernel(features, sample_res, W):
    B, G, _ = sample_res.shape
    idx = sample_res.astype(jnp.int32)
    idx_t = _tc_transpose(idx).reshape(K * NPADK)
    neigh_sum = _sc_gather_sum(features, idx_t)
    out = _tc_matmul_relu(neigh_sum, W)
    return out.reshape(B, G, D)


# speedup vs baseline: 9.7678x; 1.0022x over previous
import functools

import jax
import jax.numpy as jnp
from jax import lax
from jax.experimental import pallas as pl
from jax.experimental.pallas import tpu as pltpu
from jax.experimental.pallas import tpu_sc as plsc

D = 128
K = 25
N = 500 * 50
NC = 2
NS = 16
GSZ = 112
NBATCH = 224
PAIRB = NBATCH // NS
CORE0_BATCHES = 7
NACC = 4
NIDX = 2


def _worker_pipeline(ng, bstart, table_hbm, idx_hbm, out_hbm,
                     idx_t, accs, asems, osems, isems):
    def off_rows(i):
        off = (bstart + i) * GSZ
        return jnp.where(off > N - GSZ, N - GSZ, off)

    def stage(i):
        t = i % NIDX
        off = off_rows(i)

        def per_k(k, _):
            pltpu.async_copy(idx_hbm.at[pl.ds(k * NPADK + off, GSZ)],
                             idx_t[t].at[pl.ds(k * GSZ, GSZ)], isems[t])
            return 0

        lax.fori_loop(0, K, per_k, 0)

    def drain_stage(i):
        t = i % NIDX
        pltpu.make_async_copy(idx_hbm.at[pl.ds(0, K * GSZ)], idx_t[t],
                              isems[t]).wait()

    def k0(i):
        b = i % NACC
        pltpu.async_copy(
            table_hbm.at[idx_t[i % NIDX].at[pl.ds(0, GSZ)]],
            accs[b], asems[b])

    def wait_k0(i):
        b = i % NACC
        pltpu.make_async_copy(
            table_hbm.at[idx_t[i % NIDX].at[pl.ds(0, GSZ)]],
            accs[b], asems[b]).wait()

    def adds(i):
        b = i % NACC
        t = i % NIDX

        def per_k(k, _):
            pltpu.async_copy(
                table_hbm.at[idx_t[t].at[pl.ds(k * GSZ, GSZ)]],
                accs[b], asems[b], add=True)
            return 0

        lax.fori_loop(1, K, per_k, 0)

    def drain_adds(i):
        b = i % NACC

        def per_k(k, _):
            pltpu.make_async_copy(
                table_hbm.at[idx_t[i % NIDX].at[pl.ds(0, GSZ)]],
                accs[b], asems[b]).wait()
            return 0

        lax.fori_loop(1, K, per_k, 0)

    def flush(i):
        b = i % NACC
        pltpu.async_copy(accs[b], out_hbm.at[pl.ds(off_rows(i), GSZ)],
                         osems[b])

    def drain_flush(i):
        b = i % NACC
        pltpu.make_async_copy(accs[b], out_hbm.at[pl.ds(off_rows(i), GSZ)],
                              osems[b]).wait()

    stage(0)
    drain_stage(0)
    k0(0)

    for i in range(ng):
        wait_k0(i)
        adds(i)
        if i >= 1:
            drain_adds(i - 1)
            flush(i - 1)
        if i + 1 < ng:
            stage(i + 1)
            drain_stage(i + 1)
            if i + 1 >= NACC:
                drain_flush(i + 1 - NACC)
            k0(i + 1)

    drain_adds(ng - 1)
    flush(ng - 1)
    for j in range(max(0, ng - NACC + 1), ng):
        drain_flush(j)


def _sc_body(table_hbm, idx_hbm, out_hbm, idx_t0, idx_t1,
             acc0, acc1, acc2, acc3, as0, as1, as2, as3,
             os0, os1, os2, os3, is0, is1):
    c = lax.axis_index("c")
    s = lax.axis_index("s")
    idx_t = (idx_t0, idx_t1)
    accs = (acc0, acc1, acc2, acc3)
    asems = (as0, as1, as2, as3)
    osems = (os0, os1, os2, os3)
    isems = (is0, is1)

    a = CORE0_BATCHES
    b = PAIRB - a

    @pl.when(c == 0)
    def _():
        _worker_pipeline(a, s * a, table_hbm, idx_hbm, out_hbm,
                         idx_t, accs, asems, osems, isems)

    @pl.when(c == 1)
    def _():
        _worker_pipeline(b, NS * a + s * b, table_hbm, idx_hbm, out_hbm,
                         idx_t, accs, asems, osems, isems)


@functools.partial(
    pl.kernel,
    mesh=plsc.VectorSubcoreMesh(core_axis_name="c", subcore_axis_name="s"),
    compiler_params=pltpu.CompilerParams(use_tc_tiling_on_sc=True),
    out_type=jax.ShapeDtypeStruct((N, D), jnp.float32),
    scratch_types=(
        [pltpu.VMEM((K * GSZ,), jnp.int32) for _ in range(NIDX)]
        + [pltpu.VMEM((GSZ, D), jnp.float32) for _ in range(NACC)]
        + [pltpu.SemaphoreType.DMA for _ in range(2 * NACC + NIDX)]
    ),
)
def _sc_gather_sum(table_hbm, idx_hbm, out_hbm, *rest):
    _sc_body(table_hbm, idx_hbm, out_hbm, *rest)


NPADK = 25600


def _tr_body(x_ref, o_ref):
    xt = x_ref[...].reshape(N, K).T
    xtp = jnp.concatenate(
        [xt, jnp.zeros((K, NPADK - N), jnp.int32)], axis=1)
    o_ref[...] = xtp.reshape(K, NPADK // D, D)


_tc_transpose = pl.pallas_call(
    _tr_body,
    out_shape=jax.ShapeDtypeStruct((K, NPADK // D, D), jnp.int32),
)


TCB = 1000


def _mm_body(x_ref, w_ref, o_ref):
    o_ref[...] = jnp.maximum(
        jnp.dot(x_ref[...] * (1.0 / K), w_ref[...],
                preferred_element_type=jnp.float32),
        0.0,
    )


_tc_matmul_relu = pl.pallas_call(
    _mm_body,
    grid=(N // TCB,),
    in_specs=[
        pl.BlockSpec((TCB, D), lambda i: (i, 0)),
        pl.BlockSpec((D, D), lambda i: (0, 0)),
    ],
    out_specs=pl.BlockSpec((TCB, D), lambda i: (i, 0)),
    out_shape=jax.ShapeDtypeStruct((N, D), jnp.float32),
)


def kernel(features, sample_res, W):
    B, G, _ = sample_res.shape
    idx = sample_res.astype(jnp.int32)
    idx_t = _tc_transpose(idx).reshape(K * NPADK)
    neigh_sum = _sc_gather_sum(features, idx_t)
    out = _tc_matmul_relu(neigh_sum, W)
    return out.reshape(B, G, D)


# speedup vs baseline: 10.4216x; 1.0669x over previous
import functools

import jax
import jax.numpy as jnp
from jax import lax
from jax.experimental import pallas as pl
from jax.experimental.pallas import tpu as pltpu
from jax.experimental.pallas import tpu_sc as plsc

D = 128
K = 25
N = 500 * 50
NC = 2
NS = 16
GSZ = 112
NBATCH = 224
PAIRB = NBATCH // NS
CORE0_BATCHES = 7
NACC = 4
NIDX = 2


def _worker_pipeline(ng, bstart, table_hbm, idx_hbm, out_hbm,
                     idx_t, accs, asems, osems, isems):
    def off_rows(i):
        off = (bstart + i) * GSZ
        return jnp.where(off > N - GSZ, N - GSZ, off)

    def stage(i):
        t = i % NIDX
        off = off_rows(i)

        def per_k(k, _):
            pltpu.async_copy(idx_hbm.at[pl.ds(k * NPADK + off, GSZ)],
                             idx_t[t].at[pl.ds(k * GSZ, GSZ)], isems[t])
            return 0

        lax.fori_loop(0, K, per_k, 0)

    def drain_stage(i):
        t = i % NIDX
        pltpu.make_async_copy(idx_hbm.at[pl.ds(0, K * GSZ)], idx_t[t],
                              isems[t]).wait()

    def k0(i):
        b = i % NACC
        pltpu.async_copy(
            table_hbm.at[idx_t[i % NIDX].at[pl.ds(0, GSZ)]],
            accs[b], asems[b])

    def wait_k0(i):
        b = i % NACC
        pltpu.make_async_copy(
            table_hbm.at[idx_t[i % NIDX].at[pl.ds(0, GSZ)]],
            accs[b], asems[b]).wait()

    def adds(i):
        b = i % NACC
        t = i % NIDX

        def per_k(k, _):
            pltpu.async_copy(
                table_hbm.at[idx_t[t].at[pl.ds(k * GSZ, GSZ)]],
                accs[b], asems[b], add=True)
            return 0

        lax.fori_loop(1, K, per_k, 0)

    def drain_adds(i):
        b = i % NACC

        def per_k(k, _):
            pltpu.make_async_copy(
                table_hbm.at[idx_t[i % NIDX].at[pl.ds(0, GSZ)]],
                accs[b], asems[b]).wait()
            return 0

        lax.fori_loop(1, K, per_k, 0)

    def flush(i):
        b = i % NACC
        pltpu.async_copy(accs[b], out_hbm.at[pl.ds(off_rows(i), GSZ)],
                         osems[b])

    def drain_flush(i):
        b = i % NACC
        pltpu.make_async_copy(accs[b], out_hbm.at[pl.ds(off_rows(i), GSZ)],
                              osems[b]).wait()

    stage(0)
    drain_stage(0)
    k0(0)

    for i in range(ng):
        wait_k0(i)
        adds(i)
        if i >= 1:
            drain_adds(i - 1)
            flush(i - 1)
        if i + 1 < ng:
            stage(i + 1)
            drain_stage(i + 1)
            if i + 1 >= NACC:
                drain_flush(i + 1 - NACC)
            k0(i + 1)

    drain_adds(ng - 1)
    flush(ng - 1)
    for j in range(max(0, ng - NACC + 1), ng):
        drain_flush(j)


def _sc_body(table_hbm, idx_hbm, out_hbm, idx_t0, idx_t1,
             acc0, acc1, acc2, acc3, as0, as1, as2, as3,
             os0, os1, os2, os3, is0, is1):
    c = lax.axis_index("c")
    s = lax.axis_index("s")
    idx_t = (idx_t0, idx_t1)
    accs = (acc0, acc1, acc2, acc3)
    asems = (as0, as1, as2, as3)
    osems = (os0, os1, os2, os3)
    isems = (is0, is1)

    a = CORE0_BATCHES
    b = PAIRB - a

    @pl.when(c == 0)
    def _():
        _worker_pipeline(a, s * a, table_hbm, idx_hbm, out_hbm,
                         idx_t, accs, asems, osems, isems)

    @pl.when(c == 1)
    def _():
        _worker_pipeline(b, NS * a + s * b, table_hbm, idx_hbm, out_hbm,
                         idx_t, accs, asems, osems, isems)


@functools.partial(
    pl.kernel,
    mesh=plsc.VectorSubcoreMesh(core_axis_name="c", subcore_axis_name="s"),
    compiler_params=pltpu.CompilerParams(use_tc_tiling_on_sc=True),
    out_type=jax.ShapeDtypeStruct((N, D), jnp.float32),
    scratch_types=(
        [pltpu.VMEM((K * GSZ,), jnp.int32) for _ in range(NIDX)]
        + [pltpu.VMEM((GSZ, D), jnp.float32) for _ in range(NACC)]
        + [pltpu.SemaphoreType.DMA for _ in range(2 * NACC + NIDX)]
    ),
)
def _sc_gather_sum(table_hbm, idx_hbm, out_hbm, *rest):
    _sc_body(table_hbm, idx_hbm, out_hbm, *rest)


NPADK = 25600


def _tr_body(x_ref, o_ref):
    xt = x_ref[...].transpose(0, 2, 1).reshape(K, N)
    xtp = jnp.concatenate(
        [xt, jnp.zeros((K, NPADK - N), jnp.int32)], axis=1)
    o_ref[...] = xtp.reshape(K, NPADK // D, D)


_tc_transpose = pl.pallas_call(
    _tr_body,
    out_shape=jax.ShapeDtypeStruct((K, NPADK // D, D), jnp.int32),
)


TCB = 1000


def _mm_body(x_ref, w_ref, o_ref):
    o_ref[...] = jnp.maximum(
        jnp.dot(x_ref[...] * (1.0 / K), w_ref[...],
                preferred_element_type=jnp.float32),
        0.0,
    )


_tc_matmul_relu = pl.pallas_call(
    _mm_body,
    grid=(N // TCB,),
    in_specs=[
        pl.BlockSpec((TCB, D), lambda i: (i, 0)),
        pl.BlockSpec((D, D), lambda i: (0, 0)),
    ],
    out_specs=pl.BlockSpec((TCB, D), lambda i: (i, 0)),
    out_shape=jax.ShapeDtypeStruct((N, D), jnp.float32),
)


def kernel(features, sample_res, W):
    B, G, _ = sample_res.shape
    idx = jnp.transpose(sample_res.astype(jnp.int32), (2, 1, 0))
    idx_t = _tc_transpose(idx).reshape(K * NPADK)
    neigh_sum = _sc_gather_sum(features, idx_t)
    out = _tc_matmul_relu(neigh_sum, W)
    return out.reshape(B, G, D)
